# Initial kernel scaffold; baseline (speedup 1.0000x reference)
import jax
import jax.numpy as jnp
from jax.experimental import pallas as pl


def kernel(year, month, day, weekday, holidays, year_table, month_table, day_table, weekday_table, W_h, b_h):
    raise NotImplementedError("write your pallas kernel here")


# speedup vs baseline: 6.3778x; 6.3778x over previous
import jax
import jax.numpy as jnp
from jax.experimental import pallas as pl
from jax.experimental.pallas import tpu as pltpu

_T = 2048


def _body(y_ref, m_ref, d_ref, w_ref, hol_ref, ct_ref, w_h_ref, b_ref, out_ref):
    t = hol_ref.shape[0]
    cols = jax.lax.broadcasted_iota(jnp.int32, (t, 128), 1)
    yv = y_ref[0, 0, :][:, None]
    mv = m_ref[0, 0, :][:, None]
    dv = d_ref[0, 0, :][:, None]
    wv = w_ref[0, 0, :][:, None]
    oh = ((cols == yv).astype(jnp.float32)
          + (cols == mv + 51).astype(jnp.float32)
          + (cols == dv + 64).astype(jnp.float32)
          + (cols == wv + 96).astype(jnp.float32))
    emb = jnp.dot(oh, ct_ref[...], preferred_element_type=jnp.float32)
    lin = jax.lax.dot_general(hol_ref[...], w_h_ref[...],
                              (((1,), (1,)), ((), ())),
                              preferred_element_type=jnp.float32)
    out_ref[...] = emb + lin + b_ref[...]


def kernel(year, month, day, weekday, holidays, year_table, month_table,
           day_table, weekday_table, W_h, b_h):
    B, L = year.shape
    H = year_table.shape[1]
    K = holidays.shape[-1]
    N = B * L
    T = _T
    NB = N // T
    assert N % T == 0

    ct = jnp.concatenate([
        year_table, month_table, day_table, weekday_table,
        jnp.zeros((128 - 104, H), jnp.float32)], axis=0)

    idx_spec = pl.BlockSpec((1, 1, T), lambda i: (0, 0, i))
    full = lambda shape: pl.BlockSpec(shape, lambda i: tuple(0 for _ in shape))

    out = pl.pallas_call(
        _body,
        grid=(NB,),
        in_specs=[
            idx_spec, idx_spec, idx_spec, idx_spec,
            pl.BlockSpec((T, K), lambda i: (i, 0)),
            full((128, H)),
            full((H, K)),
            full((1, H)),
        ],
        out_specs=pl.BlockSpec((T, H), lambda i: (i, 0)),
        out_shape=jax.ShapeDtypeStruct((N, H), jnp.float32),
        compiler_params=pltpu.CompilerParams(
            dimension_semantics=("arbitrary",),
        ),
    )(
        year.reshape(1, 1, N).astype(jnp.int32),
        month.reshape(1, 1, N).astype(jnp.int32),
        day.reshape(1, 1, N).astype(jnp.int32),
        weekday.reshape(1, 1, N).astype(jnp.int32),
        holidays.reshape(N, K),
        ct,
        W_h,
        b_h.reshape(1, H),
    )
    return out.reshape(B, L, H)


# speedup vs baseline: 13.0012x; 2.0385x over previous
import jax
import jax.numpy as jnp
from jax.experimental import pallas as pl
from jax.experimental.pallas import tpu as pltpu

_TB = 256


def _body(p_ref, s_ref, hol_ref, ct_ref, w_h_ref, b_ref, out_ref):
    tb, L, K = hol_ref.shape
    H = out_ref.shape[-1]
    n = tb * 8
    shift = s_ref[...]
    ct = ct_ref[...]
    w = w_h_ref[...]
    b = b_ref[...]
    starts = list(range(0, L - 7, 8))
    if L % 8:
        starts.append(L - 8)
    for l0 in starts:
        p = p_ref[:, l0:l0 + 8][:, :, None]
        pb = jnp.broadcast_to(p, (tb, 8, H))
        cols = jax.lax.broadcasted_iota(jnp.int32, (tb, 8, H), 2)
        field = (pb >> shift) & 127
        oh = jnp.where(field == cols, 1.0, 0.0).astype(jnp.bfloat16)
        emb = jax.lax.dot_general(oh, ct, (((2,), (0,)), ((), ())),
                                  preferred_element_type=jnp.float32)
        hol = hol_ref[:, l0:l0 + 8, :].astype(jnp.bfloat16)
        lin = jax.lax.dot_general(hol, w, (((2,), (1,)), ((), ())),
                                  preferred_element_type=jnp.float32)
        out_ref[:, l0:l0 + 8, :] = emb + lin + b


def kernel(year, month, day, weekday, holidays, year_table, month_table,
           day_table, weekday_table, W_h, b_h):
    B, L = year.shape
    H = year_table.shape[1]
    K = holidays.shape[-1]
    TB = _TB
    NB = B // TB
    assert B % TB == 0

    packed = (year.astype(jnp.int32)
              | ((month.astype(jnp.int32) + 51) << 7)
              | ((day.astype(jnp.int32) + 64) << 14)
              | ((weekday.astype(jnp.int32) + 96) << 21))

    ct = jnp.concatenate([
        year_table, month_table, day_table, weekday_table,
        jnp.zeros((128 - 104, H), jnp.float32)], axis=0).astype(jnp.bfloat16)

    c = jnp.arange(H, dtype=jnp.int32)
    shift = jnp.where(c < 51, 0, jnp.where(c < 64, 7,
                      jnp.where(c < 96, 14, 21))).reshape(1, H)

    full = lambda shape: pl.BlockSpec(shape, lambda i: tuple(0 for _ in shape))

    out = pl.pallas_call(
        _body,
        grid=(NB,),
        in_specs=[
            pl.BlockSpec((TB, L), lambda i: (i, 0)),
            full((1, H)),
            pl.BlockSpec((TB, L, K), lambda i: (i, 0, 0)),
            full((128, H)),
            full((H, K)),
            full((1, H)),
        ],
        out_specs=pl.BlockSpec((TB, L, H), lambda i: (i, 0, 0)),
        out_shape=jax.ShapeDtypeStruct((B, L, H), jnp.float32),
        compiler_params=pltpu.CompilerParams(
            dimension_semantics=("arbitrary",),
        ),
    )(packed, shift, holidays, ct, W_h.astype(jnp.bfloat16), b_h.reshape(1, H))
    return out


# speedup vs baseline: 36.0648x; 2.7740x over previous
import jax
import jax.numpy as jnp
from jax.experimental import pallas as pl
from jax.experimental.pallas import tpu as pltpu


def _body(p_ref, s_ref, hol_ref, ct_ref, w_h_ref, b_ref, out_ref):
    B = out_ref.shape[1]
    H = out_ref.shape[2]
    p = p_ref[0]
    pb = jnp.broadcast_to(p, (H, B))
    shift = jnp.broadcast_to(s_ref[...], (H, B))
    rows = jax.lax.broadcasted_iota(jnp.int32, (H, B), 0)
    ohT = jnp.where(((pb >> shift) & 127) == rows, 1.0, 0.0).astype(jnp.bfloat16)
    emb = jax.lax.dot_general(ohT, ct_ref[...], (((0,), (0,)), ((), ())),
                              preferred_element_type=jnp.float32)
    hol = hol_ref[0].astype(jnp.bfloat16)
    lin = jax.lax.dot_general(hol, w_h_ref[...], (((0,), (1,)), ((), ())),
                              preferred_element_type=jnp.float32)
    out_ref[0] = emb + lin + b_ref[...]


def kernel(year, month, day, weekday, holidays, year_table, month_table,
           day_table, weekday_table, W_h, b_h):
    B, L = year.shape
    H = year_table.shape[1]
    K = holidays.shape[-1]

    packed = (year.T.astype(jnp.int32)
              | ((month.T.astype(jnp.int32) + 51) << 7)
              | ((day.T.astype(jnp.int32) + 64) << 14)
              | ((weekday.T.astype(jnp.int32) + 96) << 21))
    p3 = packed.reshape(L, 1, B)
    holT = holidays.transpose(1, 2, 0)

    ct = jnp.concatenate([
        year_table, month_table, day_table, weekday_table,
        jnp.zeros((128 - 104, H), jnp.float32)], axis=0).astype(jnp.bfloat16)

    r = jnp.arange(H, dtype=jnp.int32)
    shift = jnp.where(r < 51, 0, jnp.where(r < 64, 7,
                      jnp.where(r < 96, 14, 21))).reshape(H, 1)

    full = lambda shape: pl.BlockSpec(shape, lambda i: tuple(0 for _ in shape))

    outT = pl.pallas_call(
        _body,
        grid=(L,),
        in_specs=[
            pl.BlockSpec((1, 1, B), lambda i: (i, 0, 0)),
            full((H, 1)),
            pl.BlockSpec((1, K, B), lambda i: (i, 0, 0)),
            full((128, H)),
            full((H, K)),
            full((1, H)),
        ],
        out_specs=pl.BlockSpec((1, B, H), lambda i: (i, 0, 0)),
        out_shape=jax.ShapeDtypeStruct((L, B, H), jnp.float32),
        compiler_params=pltpu.CompilerParams(
            dimension_semantics=("arbitrary",),
            fuse_transposed_lhs_in_matmul=True,
        ),
    )(p3, shift, holT, ct, W_h.astype(jnp.bfloat16), b_h.reshape(1, H))
    return outT.transpose(1, 0, 2)


# speedup vs baseline: 38.6541x; 1.0718x over previous
import jax
import jax.numpy as jnp
from jax.experimental import pallas as pl
from jax.experimental.pallas import tpu as pltpu


def _body(p_ref, s_ref, hol_ref, yt_ref, mt_ref, dt_ref, wt_ref, w_h_ref,
          b_ref, out_ref, ct_ref, w_ref):
    B = out_ref.shape[1]
    H = out_ref.shape[2]
    i = pl.program_id(0)

    @pl.when(i == 0)
    def _init():
        ct_ref[0:51] = yt_ref[...].astype(jnp.bfloat16)
        ct_ref[51:64] = mt_ref[...].astype(jnp.bfloat16)
        ct_ref[64:96] = dt_ref[...].astype(jnp.bfloat16)
        ct_ref[96:104] = wt_ref[...].astype(jnp.bfloat16)
        ct_ref[104:128] = jnp.zeros((24, H), jnp.bfloat16)
        w_ref[...] = w_h_ref[...].astype(jnp.bfloat16)

    p = p_ref[jax.lax.rem(i, 8), :].reshape(1, B)
    pb = jnp.broadcast_to(p, (H, B))
    shift = jnp.broadcast_to(s_ref[...], (H, B))
    rows = jax.lax.broadcasted_iota(jnp.int32, (H, B), 0)
    ohT = jnp.where(((pb >> shift) & 127) == rows, 1.0, 0.0).astype(jnp.bfloat16)
    emb = jax.lax.dot_general(ohT, ct_ref[...], (((0,), (0,)), ((), ())),
                              preferred_element_type=jnp.float32)
    hol = hol_ref[0].astype(jnp.bfloat16)
    lin = jax.lax.dot_general(hol, w_ref[...], (((0,), (1,)), ((), ())),
                              preferred_element_type=jnp.float32)
    out_ref[0] = emb + lin + b_ref[...]


def kernel(year, month, day, weekday, holidays, year_table, month_table,
           day_table, weekday_table, W_h, b_h):
    B, L = year.shape
    H = year_table.shape[1]
    K = holidays.shape[-1]

    packed = (year.T.astype(jnp.int32)
              | ((month.T.astype(jnp.int32) + 51) << 7)
              | ((day.T.astype(jnp.int32) + 64) << 14)
              | ((weekday.T.astype(jnp.int32) + 96) << 21))
    holT = holidays.transpose(1, 2, 0)

    r = jnp.arange(H, dtype=jnp.int32)
    shift = jnp.where(r < 51, 0, jnp.where(r < 64, 7,
                      jnp.where(r < 96, 14, 21))).reshape(H, 1)

    full = lambda shape: pl.BlockSpec(shape, lambda i: tuple(0 for _ in shape))

    outT = pl.pallas_call(
        _body,
        grid=(L,),
        in_specs=[
            pl.BlockSpec((8, B), lambda i: (i // 8, 0)),
            full((H, 1)),
            pl.BlockSpec((1, K, B), lambda i: (i, 0, 0)),
            full((51, H)),
            full((13, H)),
            full((32, H)),
            full((8, H)),
            full((H, K)),
            full((1, H)),
        ],
        out_specs=pl.BlockSpec((1, B, H), lambda i: (i, 0, 0)),
        out_shape=jax.ShapeDtypeStruct((L, B, H), jnp.float32),
        scratch_shapes=[
            pltpu.VMEM((128, H), jnp.bfloat16),
            pltpu.VMEM((H, K), jnp.bfloat16),
        ],
        compiler_params=pltpu.CompilerParams(
            dimension_semantics=("arbitrary",),
            fuse_transposed_lhs_in_matmul=True,
        ),
    )(packed, shift, holT, year_table, month_table, day_table, weekday_table,
      W_h, b_h.reshape(1, H))
    return outT.transpose(1, 0, 2)
